# baseline (speedup 1.0000x reference)
import functools

import jax
import jax.numpy as jnp
from jax import lax
from jax.experimental import pallas as pl
from jax.experimental.pallas import tpu as pltpu
from jax.experimental.pallas import tpu_sc as plsc

B = 1024
V = 100000
K = 20
NE = B * 2 * K
NW = 32
NBUCKETS = (V + 127) // 128
G_PER_W = 112
NG = NW * G_PER_W
SLOTS_PER_W = G_PER_W * 16
PASS_G = 28
LN2 = 0.6931471805599453
SCALE = 1.0 / (B * K)


def _ln(x):
    xi = lax.bitcast_convert_type(x, jnp.int32)
    e = (xi >> 23) - 127
    m = lax.bitcast_convert_type((xi & 0x007FFFFF) | 0x3F800000, jnp.float32)
    s = (m - 1.0) / (m + 1.0)
    s2 = s * s
    poly = 1.0 + s2 * (1.0 / 3 + s2 * (1.0 / 5 + s2 * (1.0 / 7 + s2 * (1.0 / 9))))
    return e.astype(jnp.float32) * LN2 + 2.0 * s * poly


def _sc_body(table, rowslot, lcode, coffslot, out, rows, lanes, coffs, vals,
             out_v, sem):
    wid = lax.axis_index("s") * 2 + lax.axis_index("c")
    base = wid * SLOTS_PER_W

    pltpu.sync_copy(rowslot.at[pl.ds(base, SLOTS_PER_W)], rows)
    pltpu.sync_copy(lcode.at[pl.ds(base, SLOTS_PER_W)], lanes)
    pltpu.sync_copy(coffslot.at[pl.ds(base, SLOTS_PER_W)], coffs)

    lane = lax.iota(jnp.int32, 16)
    zf = jnp.zeros((16,), jnp.float32)

    accs = (zf, zf, zf)
    for p in range(4):
        pbase = p * PASS_G * 16

        def _mk(gg):
            sl = pl.multiple_of(pbase + gg * 16, 16)
            coff = pl.multiple_of(coffs[pl.ds(sl, 16)][0], 128)
            return pltpu.make_async_copy(
                table.at[:, pl.ds(coff, 128)].at[rows.at[pl.ds(sl, 16)]],
                vals.at[pl.ds(pl.multiple_of(gg * 16, 16), 16), :],
                sem)

        @pl.loop(0, PASS_G)
        def _fire(gg):
            _mk(gg).start()

        @pl.loop(0, PASS_G)
        def _drain(gg):
            _mk(gg).wait()

        def _accum(gg, carry):
            a_ko, a_en, a_rl = carry
            lc = lanes[pl.ds(pl.multiple_of(pbase + gg * 16, 16), 16)]
            is_ko = jnp.logical_and(lc >= 0, lc < 128)
            is_en = lc >= 128
            i0 = gg * 16 + lane
            x = plsc.load_gather(vals, [i0, lc & 127])
            lnv = _ln(x + 1e-8)
            a_ko = a_ko + jnp.where(is_ko, lnv, zf)
            a_en = a_en + jnp.where(is_en, lnv, zf)
            a_rl = a_rl + jnp.where(is_en, jnp.maximum(2.0 - x, 0.0), zf)
            return (a_ko, a_en, a_rl)

        accs = pl.loop(0, PASS_G, init_carry=accs)(_accum)

    a_ko, a_en, a_rl = accs
    out_v[pl.ds(0, 16)] = a_ko
    out_v[pl.ds(16, 16)] = a_en
    out_v[pl.ds(32, 16)] = a_rl
    pltpu.sync_copy(out_v, out.at[pl.ds(wid * 48, 48)])


_sc_call = functools.partial(
    pl.kernel,
    out_type=jax.ShapeDtypeStruct((NW * 48,), jnp.float32),
    mesh=plsc.VectorSubcoreMesh(core_axis_name="c", subcore_axis_name="s"),
    compiler_params=pltpu.CompilerParams(use_tc_tiling_on_sc=True,
                                         disable_bounds_checks=True,
                                         needs_layout_passes=False),
    scratch_types=[
        pltpu.VMEM((SLOTS_PER_W,), jnp.int32),
        pltpu.VMEM((SLOTS_PER_W,), jnp.int32),
        pltpu.VMEM((SLOTS_PER_W,), jnp.int32),
        pltpu.VMEM((PASS_G * 16, 128), jnp.float32),
        pltpu.VMEM((48,), jnp.float32),
        pltpu.SemaphoreType.DMA,
    ],
)(_sc_body)


def kernel(sparse_rep, ko_token_ids, en_token_ids):
    r = jnp.arange(B, dtype=jnp.int32)[:, None]
    key_ko = (ko_token_ids << 11) | (r << 1)
    key_en = (en_token_ids << 11) | (r << 1) | 1
    sk = jnp.sort(jnp.concatenate([key_ko, key_en], axis=1).reshape(-1))

    cs = sk >> 11
    bucket = cs >> 7
    first = jnp.searchsorted(bucket, bucket, side="left")
    rank = jnp.arange(NE, dtype=jnp.int32) - first.astype(jnp.int32)
    cnt = jnp.bincount(bucket, length=NBUCKETS)
    gcnt = (cnt + 15) // 16
    gbase = (jnp.cumsum(gcnt) - gcnt).astype(jnp.int32)
    group = gbase[bucket] + rank // 16
    slot = group * 16 + (rank % 16)

    rowslot = jnp.zeros((NG * 16,), jnp.int32).at[slot].set((sk >> 1) & 1023)
    lcode = jnp.full((NG * 16,), -1, jnp.int32).at[slot].set(
        (cs & 127) | ((sk & 1) << 7))
    coffg = jnp.zeros((NG,), jnp.int32).at[group].set(cs & ~127)
    coffslot = jnp.repeat(coffg, 16)

    partials = _sc_call(sparse_rep, rowslot, lcode, coffslot)
    sums = partials.reshape(NW, 3, 16).sum(axis=(0, 2))
    return jnp.stack([-sums[0] * SCALE, -sums[1] * SCALE, sums[2] * SCALE,
                      jnp.zeros((), jnp.float32)])


# speedup vs baseline: 2.6585x; 2.6585x over previous
import functools

import jax
import jax.numpy as jnp
import numpy as np
from jax import lax
from jax.experimental import pallas as pl
from jax.experimental.pallas import tpu as pltpu
from jax.experimental.pallas import tpu_sc as plsc

B = 1024
V = 100000
K = 20
NW = 32
ROWS_PER_W = B // NW
ELEMS_PER_W = ROWS_PER_W * K
CHUNK = 128
N_CHUNKS = ELEMS_PER_W // CHUNK
N_VECS = ELEMS_PER_W // 16
LN2 = 0.6931471805599453
SCALE = 1.0 / (B * K)


def _ln(x):
    xi = lax.bitcast_convert_type(x, jnp.int32)
    e = (xi >> 23) - 127
    m = lax.bitcast_convert_type((xi & 0x007FFFFF) | 0x3F800000, jnp.float32)
    s = (m - 1.0) / (m + 1.0)
    s2 = s * s
    poly = 1.0 + s2 * (1.0 / 3 + s2 * (1.0 / 5 + s2 * (1.0 / 7 + s2 * (1.0 / 9))))
    return e.astype(jnp.float32) * LN2 + 2.0 * s * poly


def _sc_body(table, ko_ids, en_ids, out, ko_idx, en_idx, ko_vals, en_vals,
             out_v, sem):
    wid = lax.axis_index("s") * 2 + lax.axis_index("c")
    base_el = wid * ELEMS_PER_W
    base_row = wid * ROWS_PER_W

    pltpu.sync_copy(ko_ids.at[pl.ds(base_el, ELEMS_PER_W)], ko_idx)
    pltpu.sync_copy(en_ids.at[pl.ds(base_el, ELEMS_PER_W)], en_idx)

    lane = lax.iota(jnp.int32, 16)
    base_off = base_row * V
    for i in range(N_VECS):
        r0 = (i * 16) // K
        cut = (r0 + 1) * K - i * 16
        if cut >= 16:
            off = base_off + r0 * V
        else:
            off = jnp.where(lane >= cut, base_off + (r0 + 1) * V,
                            base_off + r0 * V)
        ko_idx[pl.ds(i * 16, 16)] = ko_idx[pl.ds(i * 16, 16)] + off
        en_idx[pl.ds(i * 16, 16)] = en_idx[pl.ds(i * 16, 16)] + off

    copies = []
    for c in range(N_CHUNKS):
        copies.append(pltpu.async_copy(
            table.at[ko_idx.at[pl.ds(c * CHUNK, CHUNK)]],
            ko_vals.at[pl.ds(c * CHUNK, CHUNK)], sem))
        copies.append(pltpu.async_copy(
            table.at[en_idx.at[pl.ds(c * CHUNK, CHUNK)]],
            en_vals.at[pl.ds(c * CHUNK, CHUNK)], sem))
    for cp in copies:
        cp.wait()

    a_ko = jnp.zeros((16,), jnp.float32)
    a_en = jnp.zeros((16,), jnp.float32)
    a_rl = jnp.zeros((16,), jnp.float32)
    for i in range(N_VECS):
        vko = ko_vals[pl.ds(i * 16, 16)]
        ven = en_vals[pl.ds(i * 16, 16)]
        a_ko = a_ko + _ln(vko + 1e-8)
        a_en = a_en + _ln(ven + 1e-8)
        a_rl = a_rl + jnp.maximum(2.0 - ven, 0.0)

    out_v[pl.ds(0, 16)] = a_ko
    out_v[pl.ds(16, 16)] = a_en
    out_v[pl.ds(32, 16)] = a_rl
    pltpu.sync_copy(out_v, out.at[pl.ds(wid * 48, 48)])


_sc_call = functools.partial(
    pl.kernel,
    out_type=jax.ShapeDtypeStruct((NW * 48,), jnp.float32),
    mesh=plsc.VectorSubcoreMesh(core_axis_name="c", subcore_axis_name="s"),
    scratch_types=[
        pltpu.VMEM((ELEMS_PER_W,), jnp.int32),
        pltpu.VMEM((ELEMS_PER_W,), jnp.int32),
        pltpu.VMEM((ELEMS_PER_W,), jnp.float32),
        pltpu.VMEM((ELEMS_PER_W,), jnp.float32),
        pltpu.VMEM((48,), jnp.float32),
        pltpu.SemaphoreType.DMA,
    ],
)(_sc_body)


def kernel(sparse_rep, ko_token_ids, en_token_ids):
    partials = _sc_call(sparse_rep.reshape(-1),
                        ko_token_ids.reshape(-1),
                        en_token_ids.reshape(-1))
    sums = partials.reshape(NW, 3, 16).sum(axis=(0, 2))
    return jnp.stack([-sums[0] * SCALE, -sums[1] * SCALE, sums[2] * SCALE,
                      jnp.zeros((), jnp.float32)])
